# baseline (speedup 1.0000x reference)
import functools

import jax
import jax.numpy as jnp
from jax import lax
from jax.experimental import pallas as pl
from jax.experimental.pallas import tpu as pltpu
from jax.experimental.pallas import tpu_sc as plsc

_NC, _NS, _L = 2, 16, 16
_NW = _NC * _NS
_PCH = 8192


def kernel(x, table):
    batch, seq_len, dim = x.shape
    flat = seq_len * dim
    span = flat // _NW
    n_pieces = span // _PCH
    xf = x.reshape(batch, flat)
    tf = table[:seq_len].reshape(flat)

    mesh = plsc.VectorSubcoreMesh(core_axis_name="c", subcore_axis_name="s")

    @functools.partial(
        pl.kernel,
        mesh=mesh,
        out_type=jax.ShapeDtypeStruct((batch, flat), jnp.float32),
        scratch_types=(
            pltpu.VMEM((_PCH,), jnp.float32),
            pltpu.VMEM((_PCH,), jnp.float32),
            pltpu.VMEM((_PCH,), jnp.float32),
            pltpu.VMEM((_PCH,), jnp.float32),
            pltpu.VMEM((_PCH,), jnp.float32),
        ),
    )
    def k(x_hbm, t_hbm, o_hbm, tbuf, xb0, xb1, xb2, xb3):
        xbufs = (xb0, xb1, xb2, xb3)
        wid = lax.axis_index("s") * _NC + lax.axis_index("c")
        base = wid * span

        def piece(p, carry):
            off = base + p * _PCH
            pltpu.sync_copy(t_hbm.at[pl.ds(off, _PCH)], tbuf)
            for b in range(batch):
                pltpu.sync_copy(x_hbm.at[b, pl.ds(off, _PCH)], xbufs[b])

            def vec(v, c):
                tv = tbuf[pl.ds(v * _L, _L)]
                for b in range(batch):
                    plsc.addupdate(xbufs[b].at[pl.ds(v * _L, _L)], tv)
                return c

            lax.fori_loop(0, _PCH // _L, vec, 0, unroll=8)
            for b in range(batch):
                pltpu.sync_copy(xbufs[b], o_hbm.at[b, pl.ds(off, _PCH)])
            return carry

        lax.fori_loop(0, n_pieces, piece, 0)

    out = k(xf, tf)
    return out.reshape(batch, seq_len, dim)
